# baseline (speedup 1.0000x reference)
import dataclasses
import functools

import jax
import jax.numpy as jnp
from jax import lax
from jax.experimental import pallas as pl
from jax.experimental.pallas import tpu as pltpu
from jax.experimental.pallas import tpu_sc as plsc

NC = 2
NS = 16
NW = NC * NS
L = 16

CH = 128
SB = 32


def _sc_segment_sum(feat, src3, dst3, n_groups):
    n_nodes = feat.shape[0]
    iters = (n_groups + NW - 1) // NW
    per_sub = (n_nodes // NS) // 8 * 8
    last_sub = n_nodes - per_sub * (NS - 1)

    mesh = plsc.VectorSubcoreMesh(core_axis_name="c", subcore_axis_name="s")

    @functools.partial(
        pl.kernel, mesh=mesh,
        out_type=[jax.ShapeDtypeStruct((NC, n_nodes, 128), jnp.float32)],
        scratch_types=[
            pltpu.VMEM((8, SB), jnp.int32),
            pltpu.VMEM((8, SB), jnp.int32),
            pltpu.VMEM((2 * SB, 128), jnp.float32),
            pltpu.VMEM((2 * SB, 128), jnp.float32),
            pltpu.VMEM((SB, 128), jnp.float32),
            pltpu.VMEM_SHARED((n_nodes, 128), jnp.float32),
            pltpu.SemaphoreType.DMA,
            pltpu.SemaphoreType.DMA,
        ])
    def run(feat_hbm, src_hbm, dst_hbm, agg_hbm, pvs, pvd, r0, r1, r2,
            agg_s, gsem, ssem):
        c = lax.axis_index("c")
        s = lax.axis_index("s")
        wid = s * NC + c

        ZB = 2 * SB
        @pl.loop(0, ZB)
        def _(i):
            @pl.loop(0, 128, step=L)
            def _(j):
                r0[i, pl.ds(j, L)] = jnp.zeros((L,), jnp.float32)

        base = pl.multiple_of(s * per_sub, 8)
        for off in range(0, per_sub - ZB, ZB):
            pltpu.sync_copy(r0, agg_s.at[pl.ds(base + off, ZB)])
        pltpu.sync_copy(r0, agg_s.at[pl.ds(base + per_sub - ZB, ZB)])

        @pl.when(s == NS - 1)
        def _():
            off2 = pl.multiple_of(base + last_sub - ZB, 8)
            pltpu.sync_copy(r0, agg_s.at[pl.ds(off2, ZB)])

        plsc.subcore_barrier()

        slots = [r0.at[pl.ds(0, SB)], r0.at[pl.ds(SB, SB)],
                 r1.at[pl.ds(0, SB)], r1.at[pl.ds(SB, SB)], r2]

        @pl.loop(0, iters)
        def _(i):
            g = wid + NW * i

            @pl.when(g < n_groups)
            def _():
                pltpu.sync_copy(src_hbm.at[g], pvs)
                pltpu.sync_copy(dst_hbm.at[g], pvd)
                gds = [None] * 8
                sds = [None] * 8
                for k in range(3):
                    gds[k] = pltpu.async_copy(
                        feat_hbm.at[pvs.at[k]], slots[k], gsem)
                for k in range(8):
                    gds[k].wait()
                    if k < 5:
                        if k >= 2:
                            sds[k - 2].wait()
                        gds[k + 3] = pltpu.async_copy(
                            feat_hbm.at[pvs.at[k + 3]],
                            slots[(k + 3) % 5], gsem)
                    sds[k] = pltpu.async_copy(
                        slots[k % 5], agg_s.at[pvd.at[k]], ssem,
                        add=True)
                for k in range(4, 8):
                    sds[k].wait()

        plsc.subcore_barrier()

        @pl.when(s < NS - 1)
        def _():
            pltpu.sync_copy(agg_s.at[pl.ds(base, per_sub)],
                            agg_hbm.at[c].at[pl.ds(base, per_sub)])

        @pl.when(s == NS - 1)
        def _():
            pltpu.sync_copy(agg_s.at[pl.ds(base, last_sub)],
                            agg_hbm.at[c].at[pl.ds(base, last_sub)])

    return run(feat, src3, dst3)


def _sc_degree(dst1d, n_nodes):
    n_edges = dst1d.shape[0]
    per_w = n_edges // NW
    DBLK = 1024
    n_full = per_w // DBLK
    tail = per_w - n_full * DBLK

    mesh = plsc.VectorSubcoreMesh(core_axis_name="c", subcore_axis_name="s")

    cp = pltpu.CompilerParams()
    if "needs_layout_passes" in pltpu.CompilerParams.__dataclass_fields__:
        cp = dataclasses.replace(cp, needs_layout_passes=False)

    @functools.partial(
        pl.kernel, mesh=mesh, compiler_params=cp,
        out_type=jax.ShapeDtypeStruct((NW, n_nodes), jnp.float32),
        scratch_types=[
            pltpu.VMEM((DBLK,), jnp.int32),
            pltpu.VMEM((n_nodes,), jnp.float32),
        ])
    def run(dst_hbm, deg_hbm, dst_v, hist):
        c = lax.axis_index("c")
        s = lax.axis_index("s")
        wid = s * NC + c

        @pl.loop(0, n_nodes, step=L)
        def _(i):
            hist[pl.ds(i, L)] = jnp.zeros((L,), jnp.float32)

        ones = jnp.ones((L,), jnp.float32)
        e_base = pl.multiple_of(wid * per_w, 8)

        @pl.loop(0, n_full)
        def _(j):
            e0 = pl.multiple_of(e_base + j * DBLK, 8)
            pltpu.sync_copy(dst_hbm.at[pl.ds(e0, DBLK)], dst_v)

            @pl.loop(0, DBLK, step=L)
            def _(t):
                plsc.addupdate_scatter(hist, [dst_v[pl.ds(t, L)]], ones)

        if tail:
            e1 = pl.multiple_of(e_base + n_full * DBLK, 8)
            pltpu.sync_copy(dst_hbm.at[pl.ds(e1, tail)],
                            dst_v.at[pl.ds(0, tail)])

            @pl.loop(0, tail, step=L)
            def _(t):
                plsc.addupdate_scatter(hist, [dst_v[pl.ds(t, L)]], ones)

        pltpu.sync_copy(hist, deg_hbm.at[wid])

    return run(dst1d)


def _tc_sage_layer(h, agg, deg, w_self, w_neigh, b):
    n_nodes = h.shape[0]
    blk = 400
    grid = n_nodes // blk

    def body(h_ref, agg_ref, deg_ref, ws_ref, wn_ref, b_ref, o_ref):
        hb = h_ref[...]
        aggb = agg_ref[0] + agg_ref[1]
        degb = jnp.maximum(jnp.sum(deg_ref[...], axis=1), 1.0)
        hn = aggb / degb[:, None]
        acc = jnp.dot(hb, ws_ref[...], preferred_element_type=jnp.float32)
        acc += jnp.dot(hn, wn_ref[...], preferred_element_type=jnp.float32)
        o_ref[...] = jnp.maximum(acc + b_ref[...][None, :], 0.0)

    return pl.pallas_call(
        body,
        grid=(grid,),
        in_specs=[
            pl.BlockSpec((blk, 128), lambda i: (i, 0)),
            pl.BlockSpec((NC, blk, 128), lambda i: (0, i, 0)),
            pl.BlockSpec((blk, NW), lambda i: (i, 0)),
            pl.BlockSpec((128, 128), lambda i: (0, 0)),
            pl.BlockSpec((128, 128), lambda i: (0, 0)),
            pl.BlockSpec((128,), lambda i: (0,)),
        ],
        out_specs=pl.BlockSpec((blk, 128), lambda i: (i, 0)),
        out_shape=jax.ShapeDtypeStruct((n_nodes, 128), jnp.float32),
    )(h, agg, deg, w_self, w_neigh, b)


def _tc_sage_layer_pool(h, agg, deg, w_self, w_neigh, b):
    n_nodes = h.shape[0]
    blk = 400
    grid = n_nodes // blk

    def body(h_ref, agg_ref, deg_ref, ws_ref, wn_ref, b_ref, o_ref):
        i = pl.program_id(0)
        hb = h_ref[...]
        aggb = agg_ref[0] + agg_ref[1]
        degb = jnp.maximum(jnp.sum(deg_ref[...], axis=1), 1.0)
        hn = aggb / degb[:, None]
        acc = jnp.dot(hb, ws_ref[...], preferred_element_type=jnp.float32)
        acc += jnp.dot(hn, wn_ref[...], preferred_element_type=jnp.float32)
        hb2 = jnp.maximum(acc + b_ref[...][None, :], 0.0)
        part = jnp.sum(hb2, axis=0, keepdims=True)

        @pl.when(i == 0)
        def _():
            o_ref[...] = jnp.zeros_like(o_ref)

        o_ref[...] += part

        @pl.when(i == grid - 1)
        def _():
            o_ref[...] = o_ref[...] * (1.0 / n_nodes)

    return pl.pallas_call(
        body,
        grid=(grid,),
        in_specs=[
            pl.BlockSpec((blk, 128), lambda i: (i, 0)),
            pl.BlockSpec((NC, blk, 128), lambda i: (0, i, 0)),
            pl.BlockSpec((blk, NW), lambda i: (i, 0)),
            pl.BlockSpec((128, 128), lambda i: (0, 0)),
            pl.BlockSpec((128, 128), lambda i: (0, 0)),
            pl.BlockSpec((128,), lambda i: (0,)),
        ],
        out_specs=pl.BlockSpec((1, 128), lambda i: (0, 0)),
        out_shape=jax.ShapeDtypeStruct((1, 128), jnp.float32),
    )(h, agg, deg, w_self, w_neigh, b)


def kernel(x, edge_index, W_self1, W_neigh1, b1, W_self2, W_neigh2, b2):
    src1d = edge_index[0]
    dst1d = edge_index[1]
    n_groups = src1d.shape[0] // (8 * SB)
    src3 = src1d.reshape(n_groups, 8, SB)
    dst3 = dst1d.reshape(n_groups, 8, SB)

    deg = _sc_degree(dst1d, x.shape[0]).T
    (agg1,) = _sc_segment_sum(x, src3, dst3, n_groups)
    h1 = _tc_sage_layer(x, agg1, deg, W_self1, W_neigh1, b1)
    (agg2,) = _sc_segment_sum(h1, src3, dst3, n_groups)
    return _tc_sage_layer_pool(h1, agg2, deg, W_self2, W_neigh2, b2)


# speedup vs baseline: 1.0255x; 1.0255x over previous
import dataclasses
import functools

import jax
import jax.numpy as jnp
from jax import lax
from jax.experimental import pallas as pl
from jax.experimental.pallas import tpu as pltpu
from jax.experimental.pallas import tpu_sc as plsc

NC = 2
NS = 16
NW = NC * NS
L = 16

CH = 128
SB = 32


def _sc_segment_sum(feat, pairs, n_groups):
    n_nodes = feat.shape[0]
    iters = (n_groups + NW - 1) // NW
    per_sub = (n_nodes // NS) // 8 * 8
    last_sub = n_nodes - per_sub * (NS - 1)

    mesh = plsc.VectorSubcoreMesh(core_axis_name="c", subcore_axis_name="s")

    @functools.partial(
        pl.kernel, mesh=mesh,
        out_type=[jax.ShapeDtypeStruct((NC, n_nodes, 128), jnp.float32)],
        scratch_types=[
            pltpu.VMEM((16, SB), jnp.int32),
            pltpu.VMEM((2 * SB, 128), jnp.float32),
            pltpu.VMEM((2 * SB, 128), jnp.float32),
            pltpu.VMEM((SB, 128), jnp.float32),
            pltpu.VMEM_SHARED((n_nodes, 128), jnp.float32),
            pltpu.SemaphoreType.DMA,
            pltpu.SemaphoreType.DMA,
        ])
    def run(feat_hbm, pairs_hbm, agg_hbm, pv, r0, r1, r2, agg_s, gsem, ssem):
        c = lax.axis_index("c")
        s = lax.axis_index("s")
        wid = s * NC + c

        ZB = 2 * SB
        @pl.loop(0, ZB)
        def _(i):
            @pl.loop(0, 128, step=L)
            def _(j):
                r0[i, pl.ds(j, L)] = jnp.zeros((L,), jnp.float32)

        base = pl.multiple_of(s * per_sub, 8)
        for off in range(0, per_sub - ZB, ZB):
            pltpu.sync_copy(r0, agg_s.at[pl.ds(base + off, ZB)])
        pltpu.sync_copy(r0, agg_s.at[pl.ds(base + per_sub - ZB, ZB)])

        @pl.when(s == NS - 1)
        def _():
            off2 = pl.multiple_of(base + last_sub - ZB, 8)
            pltpu.sync_copy(r0, agg_s.at[pl.ds(off2, ZB)])

        plsc.subcore_barrier()

        slots = [r0.at[pl.ds(0, SB)], r0.at[pl.ds(SB, SB)],
                 r1.at[pl.ds(0, SB)], r1.at[pl.ds(SB, SB)], r2]

        @pl.loop(0, iters)
        def _(i):
            g = wid + NW * i

            @pl.when(g < n_groups)
            def _():
                pltpu.sync_copy(pairs_hbm.at[g], pv)
                gds = [None] * 8
                sds = [None] * 8
                for k in range(3):
                    gds[k] = pltpu.async_copy(
                        feat_hbm.at[pv.at[2 * k]], slots[k], gsem)
                for k in range(8):
                    gds[k].wait()
                    if k < 5:
                        if k >= 2:
                            sds[k - 2].wait()
                        gds[k + 3] = pltpu.async_copy(
                            feat_hbm.at[pv.at[2 * (k + 3)]],
                            slots[(k + 3) % 5], gsem)
                    sds[k] = pltpu.async_copy(
                        slots[k % 5], agg_s.at[pv.at[2 * k + 1]], ssem,
                        add=True)
                for k in range(4, 8):
                    sds[k].wait()

        plsc.subcore_barrier()

        @pl.when(s < NS - 1)
        def _():
            pltpu.sync_copy(agg_s.at[pl.ds(base, per_sub)],
                            agg_hbm.at[c].at[pl.ds(base, per_sub)])

        @pl.when(s == NS - 1)
        def _():
            pltpu.sync_copy(agg_s.at[pl.ds(base, last_sub)],
                            agg_hbm.at[c].at[pl.ds(base, last_sub)])

    return run(feat, pairs)


def _sc_degree(dst1d, n_nodes):
    n_edges = dst1d.shape[0]
    per_w = n_edges // NW
    DBLK = 1024
    n_full = per_w // DBLK
    tail = per_w - n_full * DBLK

    mesh = plsc.VectorSubcoreMesh(core_axis_name="c", subcore_axis_name="s")

    cp = pltpu.CompilerParams()
    if "needs_layout_passes" in pltpu.CompilerParams.__dataclass_fields__:
        cp = dataclasses.replace(cp, needs_layout_passes=False)

    @functools.partial(
        pl.kernel, mesh=mesh, compiler_params=cp,
        out_type=jax.ShapeDtypeStruct((NW, n_nodes), jnp.float32),
        scratch_types=[
            pltpu.VMEM((DBLK,), jnp.int32),
            pltpu.VMEM((n_nodes,), jnp.float32),
        ])
    def run(dst_hbm, deg_hbm, dst_v, hist):
        c = lax.axis_index("c")
        s = lax.axis_index("s")
        wid = s * NC + c

        @pl.loop(0, n_nodes, step=L)
        def _(i):
            hist[pl.ds(i, L)] = jnp.zeros((L,), jnp.float32)

        ones = jnp.ones((L,), jnp.float32)
        e_base = pl.multiple_of(wid * per_w, 8)

        @pl.loop(0, n_full)
        def _(j):
            e0 = pl.multiple_of(e_base + j * DBLK, 8)
            pltpu.sync_copy(dst_hbm.at[pl.ds(e0, DBLK)], dst_v)

            @pl.loop(0, DBLK, step=L)
            def _(t):
                plsc.addupdate_scatter(hist, [dst_v[pl.ds(t, L)]], ones)

        if tail:
            e1 = pl.multiple_of(e_base + n_full * DBLK, 8)
            pltpu.sync_copy(dst_hbm.at[pl.ds(e1, tail)],
                            dst_v.at[pl.ds(0, tail)])

            @pl.loop(0, tail, step=L)
            def _(t):
                plsc.addupdate_scatter(hist, [dst_v[pl.ds(t, L)]], ones)

        pltpu.sync_copy(hist, deg_hbm.at[wid])

    return run(dst1d)


def _tc_sage_layer(h, agg, deg, w_self, w_neigh, b):
    n_nodes = h.shape[0]
    blk = 400
    grid = n_nodes // blk

    def body(h_ref, agg_ref, deg_ref, ws_ref, wn_ref, b_ref, o_ref):
        hb = h_ref[...]
        aggb = agg_ref[0] + agg_ref[1]
        degb = jnp.maximum(jnp.sum(deg_ref[...], axis=1), 1.0)
        hn = aggb / degb[:, None]
        acc = jnp.dot(hb, ws_ref[...], preferred_element_type=jnp.float32)
        acc += jnp.dot(hn, wn_ref[...], preferred_element_type=jnp.float32)
        o_ref[...] = jnp.maximum(acc + b_ref[...][None, :], 0.0)

    return pl.pallas_call(
        body,
        grid=(grid,),
        in_specs=[
            pl.BlockSpec((blk, 128), lambda i: (i, 0)),
            pl.BlockSpec((NC, blk, 128), lambda i: (0, i, 0)),
            pl.BlockSpec((blk, NW), lambda i: (i, 0)),
            pl.BlockSpec((128, 128), lambda i: (0, 0)),
            pl.BlockSpec((128, 128), lambda i: (0, 0)),
            pl.BlockSpec((128,), lambda i: (0,)),
        ],
        out_specs=pl.BlockSpec((blk, 128), lambda i: (i, 0)),
        out_shape=jax.ShapeDtypeStruct((n_nodes, 128), jnp.float32),
    )(h, agg, deg, w_self, w_neigh, b)


def _tc_sage_layer_pool(h, agg, deg, w_self, w_neigh, b):
    n_nodes = h.shape[0]
    blk = 400
    grid = n_nodes // blk

    def body(h_ref, agg_ref, deg_ref, ws_ref, wn_ref, b_ref, o_ref):
        i = pl.program_id(0)
        hb = h_ref[...]
        aggb = agg_ref[0] + agg_ref[1]
        degb = jnp.maximum(jnp.sum(deg_ref[...], axis=1), 1.0)
        hn = aggb / degb[:, None]
        acc = jnp.dot(hb, ws_ref[...], preferred_element_type=jnp.float32)
        acc += jnp.dot(hn, wn_ref[...], preferred_element_type=jnp.float32)
        hb2 = jnp.maximum(acc + b_ref[...][None, :], 0.0)
        part = jnp.sum(hb2, axis=0, keepdims=True)

        @pl.when(i == 0)
        def _():
            o_ref[...] = jnp.zeros_like(o_ref)

        o_ref[...] += part

        @pl.when(i == grid - 1)
        def _():
            o_ref[...] = o_ref[...] * (1.0 / n_nodes)

    return pl.pallas_call(
        body,
        grid=(grid,),
        in_specs=[
            pl.BlockSpec((blk, 128), lambda i: (i, 0)),
            pl.BlockSpec((NC, blk, 128), lambda i: (0, i, 0)),
            pl.BlockSpec((blk, NW), lambda i: (i, 0)),
            pl.BlockSpec((128, 128), lambda i: (0, 0)),
            pl.BlockSpec((128, 128), lambda i: (0, 0)),
            pl.BlockSpec((128,), lambda i: (0,)),
        ],
        out_specs=pl.BlockSpec((1, 128), lambda i: (0, 0)),
        out_shape=jax.ShapeDtypeStruct((1, 128), jnp.float32),
    )(h, agg, deg, w_self, w_neigh, b)


def kernel(x, edge_index, W_self1, W_neigh1, b1, W_self2, W_neigh2, b2):
    src1d = edge_index[0]
    dst1d = edge_index[1]
    n_groups = src1d.shape[0] // (8 * SB)
    pairs = jnp.stack(
        [src1d.reshape(-1, SB), dst1d.reshape(-1, SB)], axis=1
    ).reshape(n_groups, 16, SB)

    deg = _sc_degree(dst1d, x.shape[0]).T
    (agg1,) = _sc_segment_sum(x, pairs, n_groups)
    h1 = _tc_sage_layer(x, agg1, deg, W_self1, W_neigh1, b1)
    (agg2,) = _sc_segment_sum(h1, pairs, n_groups)
    return _tc_sage_layer_pool(h1, agg2, deg, W_self2, W_neigh2, b2)
